# baseline (speedup 1.0000x reference)
import functools

import jax
import jax.numpy as jnp
from jax import lax
from jax.experimental import pallas as pl
from jax.experimental.pallas import tpu as pltpu
from jax.experimental.pallas import tpu_sc as plsc

_D = 64
_NW = 32
_G = 16
_CHUNK = 104


@functools.cache
def _make_gather(n_xrows: int, n_cols: int, n_var: int):
    r_per_w = n_xrows // _NW
    b_per_w = r_per_w * n_cols
    n_groups = r_per_w // _G
    rows_g = _G * n_cols
    n_ch = rows_g // _CHUNK
    assert rows_g % _CHUNK == 0 and n_groups % 2 == 0 and _CHUNK % 8 == 0
    mesh = plsc.VectorSubcoreMesh(core_axis_name="c", subcore_axis_name="s")

    @functools.partial(
        pl.kernel,
        mesh=mesh,
        out_type=jax.ShapeDtypeStruct((n_cols, _D, n_xrows), jnp.float32),
        scratch_types=[
            pltpu.VMEM((b_per_w,), jnp.int32),
            pltpu.VMEM((2, rows_g, _D), jnp.float32),
            pltpu.VMEM((2, n_cols, _D, _G), jnp.float32),
            pltpu.SemaphoreType.DMA,
            pltpu.SemaphoreType.DMA,
            pltpu.SemaphoreType.DMA,
            pltpu.SemaphoreType.DMA,
        ],
        compiler_params=pltpu.CompilerParams(
            use_tc_tiling_on_sc=False, needs_layout_passes=False),
    )
    def gather_kernel(idx_hbm, table_hbm, out_hbm,
                      idx_v, src_v, dst_v, gsem0, gsem1, wsem0, wsem1):
        wid = lax.axis_index("s") * 2 + lax.axis_index("c")
        i0_w = wid * r_per_w
        pltpu.sync_copy(idx_hbm.at[wid], idx_v)

        gsems = (gsem0, gsem1)
        wsems = (wsem0, wsem1)
        dvecs = [lax.iota(jnp.int32, 16) + db * 16 for db in range(_D // 16)]

        def gather_copy(g, h, k):
            return pltpu.make_async_copy(
                table_hbm.at[idx_v.at[pl.ds(g * rows_g + k * _CHUNK, _CHUNK)]],
                src_v.at[h].at[pl.ds(k * _CHUNK, _CHUNK)], gsems[h])

        def fire_g(g, h):
            for k in range(n_ch):
                gather_copy(g, h, k).start()

        def drain_g(g, h):
            for k in range(n_ch):
                gather_copy(g, h, k).wait()

        def write_copy(g, h, j):
            return pltpu.make_async_copy(
                dst_v.at[h, j],
                out_hbm.at[j, :, pl.ds(i0_w + g * _G, _G)], wsems[h])

        def fire_w(g, h):
            for j in range(n_cols):
                write_copy(g, h, j).start()

        def drain_w(g, h):
            for j in range(n_cols):
                write_copy(g, h, j).wait()

        def transpose(h):
            def qbody(q, carry):
                qs = jnp.full((16,), 0, jnp.int32) + q
                row0 = q * n_cols
                for j in range(n_cols):
                    for db in range(_D // 16):
                        v = src_v[h, row0 + j, pl.ds(db * 16, 16)]
                        plsc.store_scatter(dst_v.at[h, j], [dvecs[db], qs], v)
                return carry

            lax.fori_loop(0, _G, qbody, 0)

        fire_g(0, 0)
        fire_g(1, 1)
        drain_g(0, 0)
        transpose(0)
        fire_w(0, 0)
        fire_g(2, 0)
        drain_g(1, 1)
        transpose(1)
        fire_w(1, 1)

        def body(i, carry):
            g0 = 2 * i
            g1 = g0 + 1
            drain_w(g0 - 2, 0)
            fire_g(g1, 1)
            drain_g(g0, 0)
            transpose(0)
            drain_w(g0 - 1, 1)
            fire_w(g0, 0)
            fire_g(g0 + 2, 0)
            drain_g(g1, 1)
            transpose(1)
            fire_w(g1, 1)
            return carry

        lax.fori_loop(1, n_groups // 2 - 1, body, 0)

        gl = n_groups - 2
        drain_w(gl - 2, 0)
        fire_g(gl + 1, 1)
        drain_g(gl, 0)
        transpose(0)
        drain_w(gl - 1, 1)
        fire_w(gl, 0)
        drain_g(gl + 1, 1)
        transpose(1)
        fire_w(gl + 1, 1)
        drain_w(gl, 0)
        drain_w(gl + 1, 1)

    return gather_kernel


def kernel(x, table):
    n_xrows, n_cols = x.shape
    idx = x.astype(jnp.int32).reshape(_NW, (n_xrows // _NW) * n_cols)
    out_t = _make_gather(n_xrows, n_cols, table.shape[0])(idx, table)
    return jnp.transpose(out_t, (2, 0, 1))


# speedup vs baseline: 1.5051x; 1.5051x over previous
import functools

import jax
import jax.numpy as jnp
from jax import lax
from jax.experimental import pallas as pl
from jax.experimental.pallas import tpu as pltpu
from jax.experimental.pallas import tpu_sc as plsc

_D = 64
_NW = 32
_G = 16
_CHUNK = 104


@functools.cache
def _make_gather(n_xrows: int, n_cols: int, n_var: int):
    r_per_w = n_xrows // _NW
    b_per_w = r_per_w * n_cols
    n_groups = r_per_w // _G
    rows_g = _G * n_cols
    n_ch = rows_g // _CHUNK
    assert rows_g % _CHUNK == 0 and n_groups % 2 == 0 and _CHUNK % 8 == 0
    mesh = plsc.VectorSubcoreMesh(core_axis_name="c", subcore_axis_name="s")

    @functools.partial(
        pl.kernel,
        mesh=mesh,
        out_type=jax.ShapeDtypeStruct((n_cols, _D, n_xrows), jnp.float32),
        scratch_types=[
            pltpu.VMEM((b_per_w,), jnp.int32),
            pltpu.VMEM((2, rows_g, _D), jnp.float32),
            pltpu.VMEM((2, n_cols, _D, _G), jnp.float32),
            pltpu.SemaphoreType.DMA,
            pltpu.SemaphoreType.DMA,
            pltpu.SemaphoreType.DMA,
            pltpu.SemaphoreType.DMA,
        ],
        compiler_params=pltpu.CompilerParams(
            use_tc_tiling_on_sc=False, needs_layout_passes=False),
    )
    def gather_kernel(idx_hbm, table_hbm, out_hbm,
                      idx_v, src_v, dst_v, gsem0, gsem1, wsem0, wsem1):
        wid = lax.axis_index("s") * 2 + lax.axis_index("c")
        i0_w = wid * r_per_w
        pltpu.sync_copy(idx_hbm.at[wid], idx_v)

        gsems = (gsem0, gsem1)
        wsems = (wsem0, wsem1)
        dvecs = [lax.iota(jnp.int32, 16) + db * 16 for db in range(_D // 16)]

        def gather_copy(g, h, k):
            return pltpu.make_async_copy(
                table_hbm.at[idx_v.at[pl.ds(g * rows_g + k * _CHUNK, _CHUNK)]],
                src_v.at[h].at[pl.ds(k * _CHUNK, _CHUNK)], gsems[h])

        def fire_g(g, h):
            for k in range(n_ch):
                gather_copy(g, h, k).start()

        def drain_g(g, h):
            for k in range(n_ch):
                gather_copy(g, h, k).wait()

        def write_copy(g, h, j):
            return pltpu.make_async_copy(
                dst_v.at[h, j],
                out_hbm.at[j, :, pl.ds(i0_w + g * _G, _G)], wsems[h])

        def fire_w(g, h):
            for j in range(n_cols):
                write_copy(g, h, j).start()

        def drain_w(g, h):
            for j in range(n_cols):
                write_copy(g, h, j).wait()

        def transpose(h):
            @plsc.parallel_loop(0, _G, unroll=2)
            def qbody(q):
                qs = jnp.full((16,), 0, jnp.int32) + q
                row0 = q * n_cols
                for j in range(n_cols):
                    for db in range(_D // 16):
                        v = src_v[h, row0 + j, pl.ds(db * 16, 16)]
                        plsc.store_scatter(dst_v.at[h, j], [dvecs[db], qs], v)

        fire_g(0, 0)
        fire_g(1, 1)
        drain_g(0, 0)
        transpose(0)
        fire_w(0, 0)
        fire_g(2, 0)
        drain_g(1, 1)
        transpose(1)
        fire_w(1, 1)

        def body(i, carry):
            g0 = 2 * i
            g1 = g0 + 1
            drain_w(g0 - 2, 0)
            fire_g(g1, 1)
            drain_g(g0, 0)
            transpose(0)
            drain_w(g0 - 1, 1)
            fire_w(g0, 0)
            fire_g(g0 + 2, 0)
            drain_g(g1, 1)
            transpose(1)
            fire_w(g1, 1)
            return carry

        lax.fori_loop(1, n_groups // 2 - 1, body, 0)

        gl = n_groups - 2
        drain_w(gl - 2, 0)
        fire_g(gl + 1, 1)
        drain_g(gl, 0)
        transpose(0)
        drain_w(gl - 1, 1)
        fire_w(gl, 0)
        drain_g(gl + 1, 1)
        transpose(1)
        fire_w(gl + 1, 1)
        drain_w(gl, 0)
        drain_w(gl + 1, 1)

    return gather_kernel


def kernel(x, table):
    n_xrows, n_cols = x.shape
    idx = x.astype(jnp.int32).reshape(_NW, (n_xrows // _NW) * n_cols)
    out_t = _make_gather(n_xrows, n_cols, table.shape[0])(idx, table)
    return jnp.transpose(out_t, (2, 0, 1))
